# baseline (speedup 1.0000x reference)
import functools

import jax
import jax.numpy as jnp
from jax import lax
from jax.experimental import pallas as pl
from jax.experimental.pallas import tpu as pltpu
from jax.experimental.pallas import tpu_sc as plsc

NC = 2
NS = 16
L = 16
NW = NC * NS
NPIX = 1024 * 1024
PER_W = NPIX // NW
ROWS_W = 1024 // NW
HALF_ROWS = ROWS_W // 2
HALF_PIX = HALF_ROWS * 1024
NSUB = 4


def _hist_body(p_hbm, t_hbm, out_hbm, p_v, t_v, h_v, h2_v, s0, s1):
    wid = lax.axis_index("s") * NC + lax.axis_index("c")
    rb = wid * ROWS_W
    c_p0 = pltpu.async_copy(
        p_hbm.at[pl.ds(rb, HALF_ROWS)], p_v.at[pl.ds(0, HALF_ROWS)], s0
    )
    c_t0 = pltpu.async_copy(
        t_hbm.at[pl.ds(rb, HALF_ROWS)], t_v.at[pl.ds(0, HALF_ROWS)], s0
    )
    c_p1 = pltpu.async_copy(
        p_hbm.at[pl.ds(rb + HALF_ROWS, HALF_ROWS)],
        p_v.at[pl.ds(HALF_ROWS, HALF_ROWS)],
        s1,
    )
    c_t1 = pltpu.async_copy(
        t_hbm.at[pl.ds(rb + HALF_ROWS, HALF_ROWS)],
        t_v.at[pl.ds(HALF_ROWS, HALF_ROWS)],
        s1,
    )

    zeros = jnp.zeros((L,), jnp.float32)
    for b in range(256 * NSUB // L):
        h_v[pl.ds(b * L, L)] = zeros

    ones = jnp.ones((L,), jnp.float32)
    gf = (lax.iota(jnp.int32, L) & (NSUB - 1)).astype(jnp.float32)

    def scatter_range(lo, hi):
        @plsc.parallel_loop(lo, hi, step=L, unroll=32)
        def _scatter(i):
            r = i >> 10
            c = i & 1023
            p = p_v[r, pl.ds(c, L)]
            t = t_v[r, pl.ds(c, L)]
            key = ((p * 16.0 + t) * float(NSUB) + gf).astype(jnp.int32)
            plsc.addupdate_scatter(h_v, [key], ones)

    c_p0.wait()
    c_t0.wait()
    scatter_range(0, HALF_PIX)
    c_p1.wait()
    c_t1.wait()
    scatter_range(HALF_PIX, PER_W)

    iot = lax.iota(jnp.int32, L)
    for i in range(16):
        acc = zeros
        for g in range(NSUB):
            idx = (i * 16 + iot) * NSUB + g
            acc = acc + plsc.load_gather(h_v, [idx])
        h2_v[i, :] = acc
    pltpu.sync_copy(h2_v, out_hbm.at[wid])


def _make_hist_kernel():
    mesh = plsc.VectorSubcoreMesh(core_axis_name="c", subcore_axis_name="s")
    return functools.partial(
        pl.kernel,
        mesh=mesh,
        out_type=jax.ShapeDtypeStruct((NW, 16, 16), jnp.float32),
        scratch_types=[
            pltpu.VMEM((ROWS_W, 1024), jnp.float32),
            pltpu.VMEM((ROWS_W, 1024), jnp.float32),
            pltpu.VMEM((256 * NSUB,), jnp.float32),
            pltpu.VMEM((16, 16), jnp.float32),
            pltpu.SemaphoreType.DMA,
            pltpu.SemaphoreType.DMA,
        ],
        compiler_params=pltpu.CompilerParams(
            needs_layout_passes=False, use_tc_tiling_on_sc=True
        ),
    )(_hist_body)


def _finalize_body(part_ref, out_ref):
    H = jnp.sum(part_ref[...], axis=0)
    cp = jnp.sum(H, axis=1, keepdims=True)
    ct = jnp.sum(H, axis=0, keepdims=True)
    union = cp + ct - H
    iou = H / jnp.maximum(union, 1.0)
    jcol = lax.broadcasted_iota(jnp.int32, (16, 16), 1)
    iou = jnp.where(jcol >= 1, iou, 0.0)
    max_iou = jnp.max(iou, axis=1, keepdims=True)
    irow = lax.broadcasted_iota(jnp.int32, (16, 1), 0)
    present = jnp.logical_and(cp > 0.0, irow >= 1)
    total = jnp.sum(jnp.where(present, 1.0 - max_iou, 0.0))
    ninst = jnp.sum(jnp.where(present, 1.0, 0.0))
    sum_p = jnp.sum(cp * irow.astype(jnp.float32))
    sum_t = jnp.sum(ct * lax.broadcasted_iota(jnp.int32, (1, 16), 1).astype(jnp.float32))
    loss = total + sum_p / 1e12 + sum_t / 1e12
    out_ref[...] = jnp.where(ninst == 0.0, jnp.float32(0.0), loss).reshape(1, 1)


@jax.jit
def _run(pred_mask, true_mask):
    part = _make_hist_kernel()(pred_mask, true_mask)
    out = pl.pallas_call(
        _finalize_body,
        out_shape=jax.ShapeDtypeStruct((1, 1), jnp.float32),
    )(part)
    return out[0, 0]


def kernel(pred_mask, true_mask):
    return _run(pred_mask, true_mask)


# speedup vs baseline: 1.0246x; 1.0246x over previous
import functools

import jax
import jax.numpy as jnp
from jax import lax
from jax.experimental import pallas as pl
from jax.experimental.pallas import tpu as pltpu
from jax.experimental.pallas import tpu_sc as plsc

NC = 2
NS = 16
L = 16
NW = NC * NS
NPIX = 1024 * 1024
PER_W = NPIX // NW
ROWS_W = 1024 // NW
HALF_ROWS = ROWS_W // 2
HALF_PIX = HALF_ROWS * 1024
NSUB = 4


def _hist_body(p_hbm, t_hbm, out_hbm, p_v, t_v, h_v, h2_v, s0, s1):
    wid = lax.axis_index("s") * NC + lax.axis_index("c")
    rb = wid * ROWS_W
    c_p0 = pltpu.async_copy(
        p_hbm.at[pl.ds(rb, HALF_ROWS)], p_v.at[pl.ds(0, HALF_ROWS)], s0
    )
    c_t0 = pltpu.async_copy(
        t_hbm.at[pl.ds(rb, HALF_ROWS)], t_v.at[pl.ds(0, HALF_ROWS)], s0
    )
    c_p1 = pltpu.async_copy(
        p_hbm.at[pl.ds(rb + HALF_ROWS, HALF_ROWS)],
        p_v.at[pl.ds(HALF_ROWS, HALF_ROWS)],
        s1,
    )
    c_t1 = pltpu.async_copy(
        t_hbm.at[pl.ds(rb + HALF_ROWS, HALF_ROWS)],
        t_v.at[pl.ds(HALF_ROWS, HALF_ROWS)],
        s1,
    )

    zeros = jnp.zeros((L,), jnp.float32)
    for b in range(256 * NSUB // L):
        h_v[pl.ds(b * L, L)] = zeros

    ones = jnp.ones((L,), jnp.float32)
    gf = (lax.iota(jnp.int32, L) & (NSUB - 1)).astype(jnp.float32)

    def scatter_range(lo, hi):
        @plsc.parallel_loop(lo, hi, step=L, unroll=16)
        def _scatter(i):
            r = i >> 10
            c = i & 1023
            p = p_v[r, pl.ds(c, L)]
            t = t_v[r, pl.ds(c, L)]
            key = ((p * 16.0 + t) * float(NSUB) + gf).astype(jnp.int32)
            plsc.addupdate_scatter(h_v, [key], ones)

    c_p0.wait()
    c_t0.wait()
    scatter_range(0, HALF_PIX)
    c_p1.wait()
    c_t1.wait()
    scatter_range(HALF_PIX, PER_W)

    iot = lax.iota(jnp.int32, L)
    for i in range(16):
        acc = zeros
        for g in range(NSUB):
            idx = (i * 16 + iot) * NSUB + g
            acc = acc + plsc.load_gather(h_v, [idx])
        h2_v[i, :] = acc
    pltpu.sync_copy(h2_v, out_hbm.at[wid])


def _make_hist_kernel():
    mesh = plsc.VectorSubcoreMesh(core_axis_name="c", subcore_axis_name="s")
    return functools.partial(
        pl.kernel,
        mesh=mesh,
        out_type=jax.ShapeDtypeStruct((NW, 16, 16), jnp.float32),
        scratch_types=[
            pltpu.VMEM((ROWS_W, 1024), jnp.float32),
            pltpu.VMEM((ROWS_W, 1024), jnp.float32),
            pltpu.VMEM((256 * NSUB,), jnp.float32),
            pltpu.VMEM((16, 16), jnp.float32),
            pltpu.SemaphoreType.DMA,
            pltpu.SemaphoreType.DMA,
        ],
        compiler_params=pltpu.CompilerParams(
            needs_layout_passes=False, use_tc_tiling_on_sc=True
        ),
    )(_hist_body)


def _finalize_body(part_ref, out_ref):
    H = jnp.sum(part_ref[...], axis=0)
    cp = jnp.sum(H, axis=1, keepdims=True)
    ct = jnp.sum(H, axis=0, keepdims=True)
    union = cp + ct - H
    iou = H / jnp.maximum(union, 1.0)
    jcol = lax.broadcasted_iota(jnp.int32, (16, 16), 1)
    iou = jnp.where(jcol >= 1, iou, 0.0)
    max_iou = jnp.max(iou, axis=1, keepdims=True)
    irow = lax.broadcasted_iota(jnp.int32, (16, 1), 0)
    present = jnp.logical_and(cp > 0.0, irow >= 1)
    total = jnp.sum(jnp.where(present, 1.0 - max_iou, 0.0))
    ninst = jnp.sum(jnp.where(present, 1.0, 0.0))
    sum_p = jnp.sum(cp * irow.astype(jnp.float32))
    sum_t = jnp.sum(ct * lax.broadcasted_iota(jnp.int32, (1, 16), 1).astype(jnp.float32))
    loss = total + sum_p / 1e12 + sum_t / 1e12
    out_ref[...] = jnp.where(ninst == 0.0, jnp.float32(0.0), loss).reshape(1, 1)


@jax.jit
def _run(pred_mask, true_mask):
    part = _make_hist_kernel()(pred_mask, true_mask)
    out = pl.pallas_call(
        _finalize_body,
        out_shape=jax.ShapeDtypeStruct((1, 1), jnp.float32),
    )(part)
    return out[0, 0]


def kernel(pred_mask, true_mask):
    return _run(pred_mask, true_mask)
